# Initial kernel scaffold; baseline (speedup 1.0000x reference)
import jax
import jax.numpy as jnp
from jax.experimental import pallas as pl


def kernel(x, W):
    raise NotImplementedError("write your pallas kernel here")


# speedup vs baseline: 1.2363x; 1.2363x over previous
import functools

import jax
import jax.numpy as jnp
from jax.experimental import pallas as pl

AUX_COEF = 0.01
Z_COEF = 0.001
BLK = 512


def _router_body(x_ref, w_ref, i0_ref, i1_ref, w0_ref, w1_ref,
                 cnt_ref, ps_ref, lse_ref, *, n_experts):
    logits = jnp.dot(x_ref[...], w_ref[...],
                     preferred_element_type=jnp.float32)
    iota = jax.lax.broadcasted_iota(jnp.int32, logits.shape, 1)

    m0 = jnp.max(logits, axis=1, keepdims=True)
    i0 = jnp.min(jnp.where(logits == m0, iota, n_experts), axis=1,
                 keepdims=True)
    masked = jnp.where(iota == i0, jnp.float32(-1e30), logits)
    m1 = jnp.max(masked, axis=1, keepdims=True)
    i1 = jnp.min(jnp.where(masked == m1, iota, n_experts), axis=1,
                 keepdims=True)

    e1 = jnp.exp(m1 - m0)
    denom = 1.0 + e1
    w0_ref[...] = 1.0 / denom
    w1_ref[...] = e1 / denom
    i0_ref[...] = i0
    i1_ref[...] = i1

    ex = jnp.exp(logits - m0)
    ssum = jnp.sum(ex, axis=1, keepdims=True)
    ps_blk = jnp.sum(ex / ssum, axis=0)[None, :]
    lse_blk = jnp.sum(m0 + jnp.log(ssum), keepdims=True)
    one_hot = ((iota == i0).astype(jnp.float32)
               + (iota == i1).astype(jnp.float32))
    cnt_blk = jnp.sum(one_hot, axis=0)[None, :]

    @pl.when(pl.program_id(0) == 0)
    def _init():
        cnt_ref[...] = jnp.zeros_like(cnt_ref)
        ps_ref[...] = jnp.zeros_like(ps_ref)
        lse_ref[...] = jnp.zeros_like(lse_ref)

    cnt_ref[...] += cnt_blk
    ps_ref[...] += ps_blk
    lse_ref[...] += lse_blk


def kernel(x, W):
    B, S, D = x.shape
    E = W.shape[1]
    N = B * S
    x2 = x.reshape(N, D)

    body = functools.partial(_router_body, n_experts=E)
    i0, i1, w0, w1, cnt, ps, lse = pl.pallas_call(
        body,
        grid=(N // BLK,),
        in_specs=[
            pl.BlockSpec((BLK, D), lambda i: (i, 0)),
            pl.BlockSpec((D, E), lambda i: (0, 0)),
        ],
        out_specs=[
            pl.BlockSpec((BLK, 1), lambda i: (i, 0)),
            pl.BlockSpec((BLK, 1), lambda i: (i, 0)),
            pl.BlockSpec((BLK, 1), lambda i: (i, 0)),
            pl.BlockSpec((BLK, 1), lambda i: (i, 0)),
            pl.BlockSpec((1, E), lambda i: (0, 0)),
            pl.BlockSpec((1, E), lambda i: (0, 0)),
            pl.BlockSpec((1, 1), lambda i: (0, 0)),
        ],
        out_shape=[
            jax.ShapeDtypeStruct((N, 1), jnp.int32),
            jax.ShapeDtypeStruct((N, 1), jnp.int32),
            jax.ShapeDtypeStruct((N, 1), jnp.float32),
            jax.ShapeDtypeStruct((N, 1), jnp.float32),
            jax.ShapeDtypeStruct((1, E), jnp.float32),
            jax.ShapeDtypeStruct((1, E), jnp.float32),
            jax.ShapeDtypeStruct((1, 1), jnp.float32),
        ],
    )(x2, W)

    idx = jnp.concatenate([i0, i1], axis=1).reshape(B, S, 2)
    wts = jnp.concatenate([w0, w1], axis=1).reshape(B, S, 2)
    tokens_per_expert = cnt[0] / N
    router_prob_per_expert = ps[0] / N
    balance_loss = jnp.sum(tokens_per_expert * router_prob_per_expert) * E
    z_loss = (lse[0, 0] / N) ** 2
    return (idx, wts, balance_loss * AUX_COEF, z_loss * Z_COEF,
            tokens_per_expert)


# speedup vs baseline: 1.3622x; 1.1018x over previous
import functools

import jax
import jax.numpy as jnp
from jax.experimental import pallas as pl

AUX_COEF = 0.01
Z_COEF = 0.001
BLK = 1024


def _router_body(x_ref, w_ref, i0_ref, i1_ref, w0_ref, w1_ref,
                 cnt_ref, ps_ref, lse_ref, *, n_experts):
    logits = jnp.dot(x_ref[...], w_ref[...],
                     preferred_element_type=jnp.float32)
    iota = jax.lax.broadcasted_iota(jnp.int32, logits.shape, 1)

    m0 = jnp.max(logits, axis=1, keepdims=True)
    i0 = jnp.min(jnp.where(logits == m0, iota, n_experts), axis=1,
                 keepdims=True)
    masked = jnp.where(iota == i0, jnp.float32(-1e30), logits)
    m1 = jnp.max(masked, axis=1, keepdims=True)
    i1 = jnp.min(jnp.where(masked == m1, iota, n_experts), axis=1,
                 keepdims=True)

    e1 = jnp.exp(m1 - m0)
    denom = 1.0 + e1
    w0_ref[...] = 1.0 / denom
    w1_ref[...] = e1 / denom
    i0_ref[...] = i0
    i1_ref[...] = i1

    ex = jnp.exp(logits - m0)
    ssum = jnp.sum(ex, axis=1, keepdims=True)
    ps_blk = jnp.sum(ex / ssum, axis=0)[None, :]
    lse_blk = jnp.sum(m0 + jnp.log(ssum), keepdims=True)
    one_hot = ((iota == i0).astype(jnp.float32)
               + (iota == i1).astype(jnp.float32))
    cnt_blk = jnp.sum(one_hot, axis=0)[None, :]

    @pl.when(pl.program_id(0) == 0)
    def _init():
        cnt_ref[...] = jnp.zeros_like(cnt_ref)
        ps_ref[...] = jnp.zeros_like(ps_ref)
        lse_ref[...] = jnp.zeros_like(lse_ref)

    cnt_ref[...] += cnt_blk
    ps_ref[...] += ps_blk
    lse_ref[...] += lse_blk


def kernel(x, W):
    B, S, D = x.shape
    E = W.shape[1]
    N = B * S
    x2 = x.reshape(N, D)

    body = functools.partial(_router_body, n_experts=E)
    i0, i1, w0, w1, cnt, ps, lse = pl.pallas_call(
        body,
        grid=(N // BLK,),
        in_specs=[
            pl.BlockSpec((BLK, D), lambda i: (i, 0)),
            pl.BlockSpec((D, E), lambda i: (0, 0)),
        ],
        out_specs=[
            pl.BlockSpec((BLK, 1), lambda i: (i, 0)),
            pl.BlockSpec((BLK, 1), lambda i: (i, 0)),
            pl.BlockSpec((BLK, 1), lambda i: (i, 0)),
            pl.BlockSpec((BLK, 1), lambda i: (i, 0)),
            pl.BlockSpec((1, E), lambda i: (0, 0)),
            pl.BlockSpec((1, E), lambda i: (0, 0)),
            pl.BlockSpec((1, 1), lambda i: (0, 0)),
        ],
        out_shape=[
            jax.ShapeDtypeStruct((N, 1), jnp.int32),
            jax.ShapeDtypeStruct((N, 1), jnp.int32),
            jax.ShapeDtypeStruct((N, 1), jnp.float32),
            jax.ShapeDtypeStruct((N, 1), jnp.float32),
            jax.ShapeDtypeStruct((1, E), jnp.float32),
            jax.ShapeDtypeStruct((1, E), jnp.float32),
            jax.ShapeDtypeStruct((1, 1), jnp.float32),
        ],
    )(x2, W)

    idx = jnp.concatenate([i0, i1], axis=1).reshape(B, S, 2)
    wts = jnp.concatenate([w0, w1], axis=1).reshape(B, S, 2)
    tokens_per_expert = cnt[0] / N
    router_prob_per_expert = ps[0] / N
    balance_loss = jnp.sum(tokens_per_expert * router_prob_per_expert) * E
    z_loss = (lse[0, 0] / N) ** 2
    return (idx, wts, balance_loss * AUX_COEF, z_loss * Z_COEF,
            tokens_per_expert)


# speedup vs baseline: 1.4023x; 1.0295x over previous
import functools

import jax
import jax.numpy as jnp
from jax.experimental import pallas as pl

AUX_COEF = 0.01
Z_COEF = 0.001
BLK = 2048


def _router_body(x_ref, w_ref, i0_ref, i1_ref, w0_ref, w1_ref,
                 cnt_ref, ps_ref, lse_ref, *, n_experts):
    logits = jnp.dot(x_ref[...], w_ref[...],
                     preferred_element_type=jnp.float32)
    iota = jax.lax.broadcasted_iota(jnp.int32, logits.shape, 1)

    m0 = jnp.max(logits, axis=1, keepdims=True)
    i0 = jnp.min(jnp.where(logits == m0, iota, n_experts), axis=1,
                 keepdims=True)
    masked = jnp.where(iota == i0, jnp.float32(-1e30), logits)
    m1 = jnp.max(masked, axis=1, keepdims=True)
    i1 = jnp.min(jnp.where(masked == m1, iota, n_experts), axis=1,
                 keepdims=True)

    e1 = jnp.exp(m1 - m0)
    denom = 1.0 + e1
    w0_ref[...] = 1.0 / denom
    w1_ref[...] = e1 / denom
    i0_ref[...] = i0
    i1_ref[...] = i1

    ex = jnp.exp(logits - m0)
    ssum = jnp.sum(ex, axis=1, keepdims=True)
    ps_blk = jnp.sum(ex / ssum, axis=0)[None, :]
    lse_blk = jnp.sum(m0 + jnp.log(ssum), keepdims=True)
    one_hot = ((iota == i0).astype(jnp.float32)
               + (iota == i1).astype(jnp.float32))
    cnt_blk = jnp.sum(one_hot, axis=0)[None, :]

    @pl.when(pl.program_id(0) == 0)
    def _init():
        cnt_ref[...] = jnp.zeros_like(cnt_ref)
        ps_ref[...] = jnp.zeros_like(ps_ref)
        lse_ref[...] = jnp.zeros_like(lse_ref)

    cnt_ref[...] += cnt_blk
    ps_ref[...] += ps_blk
    lse_ref[...] += lse_blk


def kernel(x, W):
    B, S, D = x.shape
    E = W.shape[1]
    N = B * S
    x2 = x.reshape(N, D)

    body = functools.partial(_router_body, n_experts=E)
    i0, i1, w0, w1, cnt, ps, lse = pl.pallas_call(
        body,
        grid=(N // BLK,),
        in_specs=[
            pl.BlockSpec((BLK, D), lambda i: (i, 0)),
            pl.BlockSpec((D, E), lambda i: (0, 0)),
        ],
        out_specs=[
            pl.BlockSpec((BLK, 1), lambda i: (i, 0)),
            pl.BlockSpec((BLK, 1), lambda i: (i, 0)),
            pl.BlockSpec((BLK, 1), lambda i: (i, 0)),
            pl.BlockSpec((BLK, 1), lambda i: (i, 0)),
            pl.BlockSpec((1, E), lambda i: (0, 0)),
            pl.BlockSpec((1, E), lambda i: (0, 0)),
            pl.BlockSpec((1, 1), lambda i: (0, 0)),
        ],
        out_shape=[
            jax.ShapeDtypeStruct((N, 1), jnp.int32),
            jax.ShapeDtypeStruct((N, 1), jnp.int32),
            jax.ShapeDtypeStruct((N, 1), jnp.float32),
            jax.ShapeDtypeStruct((N, 1), jnp.float32),
            jax.ShapeDtypeStruct((1, E), jnp.float32),
            jax.ShapeDtypeStruct((1, E), jnp.float32),
            jax.ShapeDtypeStruct((1, 1), jnp.float32),
        ],
    )(x2, W)

    idx = jnp.concatenate([i0, i1], axis=1).reshape(B, S, 2)
    wts = jnp.concatenate([w0, w1], axis=1).reshape(B, S, 2)
    tokens_per_expert = cnt[0] / N
    router_prob_per_expert = ps[0] / N
    balance_loss = jnp.sum(tokens_per_expert * router_prob_per_expert) * E
    z_loss = (lse[0, 0] / N) ** 2
    return (idx, wts, balance_loss * AUX_COEF, z_loss * Z_COEF,
            tokens_per_expert)
